# Initial kernel scaffold; baseline (speedup 1.0000x reference)
import jax
import jax.numpy as jnp
from jax.experimental import pallas as pl


def kernel(indices, table):
    raise NotImplementedError("write your pallas kernel here")


# speedup vs baseline: 1.0317x; 1.0317x over previous
import jax
import jax.numpy as jnp
from jax import lax
from jax.experimental import pallas as pl
from jax.experimental.pallas import tpu as pltpu
from jax.experimental.pallas import tpu_sc as plsc

VOCAB = 100004
DIM = 200
BATCH = 4096
SEQ = 50

NC = 2
NS = 16
NW = NC * NS

B_TOTAL = BATCH * SEQ
ROWS_PER_W = B_TOTAL // NW
CHUNK = 128
N_CHUNKS = ROWS_PER_W // CHUNK


def _body(table_hbm, idx_hbm, out_hbm, idx_v, rows_v, sem):
    c = lax.axis_index("c")
    s = lax.axis_index("s")
    wid = s * NC + c
    base = wid * ROWS_PER_W

    pltpu.sync_copy(idx_hbm.at[wid], idx_v)

    @pl.loop(0, N_CHUNKS)
    def _chunk(i):
        gather = pltpu.make_async_copy(table_hbm.at[idx_v.at[i]], rows_v, sem)
        gather.start()
        gather.wait()
        pltpu.sync_copy(rows_v, out_hbm.at[pl.ds(base + i * CHUNK, CHUNK)])


@jax.jit
def _embed(idx3, table):
    mesh = plsc.VectorSubcoreMesh(core_axis_name="c", subcore_axis_name="s")
    f = pl.kernel(
        _body,
        out_type=jax.ShapeDtypeStruct((B_TOTAL, DIM), jnp.float32),
        mesh=mesh,
        scratch_types=[
            pltpu.VMEM((N_CHUNKS, CHUNK), jnp.int32),
            pltpu.VMEM((CHUNK, DIM), jnp.float32),
            pltpu.SemaphoreType.DMA,
        ],
        compiler_params=pltpu.CompilerParams(use_tc_tiling_on_sc=False),
    )
    return f(table, idx3)


def kernel(indices, table):
    idx3 = indices.reshape(NW, N_CHUNKS, CHUNK).astype(jnp.int32)
    out = _embed(idx3, table)
    return out.reshape(BATCH, SEQ, DIM)


# speedup vs baseline: 1.0678x; 1.0349x over previous
import jax
import jax.numpy as jnp
from jax import lax
from jax.experimental import pallas as pl
from jax.experimental.pallas import tpu as pltpu
from jax.experimental.pallas import tpu_sc as plsc

VOCAB = 100004
DIM = 200
BATCH = 4096
SEQ = 50

NC = 2
NS = 16
NW = NC * NS

B_TOTAL = BATCH * SEQ
ROWS_PER_W = B_TOTAL // NW
CHUNK = 100
N_CHUNKS = ROWS_PER_W // CHUNK
NBUF = 4
N_OUTER = N_CHUNKS // NBUF


def _body(table_hbm, idx_hbm, out_hbm, idx_v, *scratch):
    bufs = scratch[:NBUF]
    gsems = scratch[NBUF:2 * NBUF]
    wsems = scratch[2 * NBUF:]

    c = lax.axis_index("c")
    s = lax.axis_index("s")
    wid = s * NC + c
    base = wid * ROWS_PER_W

    pltpu.sync_copy(idx_hbm.at[wid], idx_v)

    for b in range(NBUF):
        pltpu.make_async_copy(
            table_hbm.at[idx_v.at[b]], bufs[b], gsems[b]).start()

    @pl.loop(0, N_OUTER)
    def _outer(o):
        c0 = o * NBUF
        for b in range(NBUF):
            ci = c0 + b
            pltpu.make_async_copy(
                table_hbm.at[idx_v.at[ci]], bufs[b], gsems[b]).wait()
            wb = pltpu.make_async_copy(
                bufs[b], out_hbm.at[pl.ds(base + ci * CHUNK, CHUNK)],
                wsems[b])
            wb.start()
            wb.wait()

            @pl.when(ci + NBUF < N_CHUNKS)
            def _():
                pltpu.make_async_copy(
                    table_hbm.at[idx_v.at[ci + NBUF]], bufs[b],
                    gsems[b]).start()


@jax.jit
def _embed(idx3, table):
    mesh = plsc.VectorSubcoreMesh(core_axis_name="c", subcore_axis_name="s")
    f = pl.kernel(
        _body,
        out_type=jax.ShapeDtypeStruct((B_TOTAL, DIM), jnp.float32),
        mesh=mesh,
        scratch_types=[
            pltpu.VMEM((N_CHUNKS, CHUNK), jnp.int32),
            *[pltpu.VMEM((CHUNK, DIM), jnp.float32) for _ in range(NBUF)],
            *[pltpu.SemaphoreType.DMA for _ in range(NBUF)],
            *[pltpu.SemaphoreType.DMA for _ in range(NBUF)],
        ],
        compiler_params=pltpu.CompilerParams(use_tc_tiling_on_sc=False),
    )
    return f(table, idx3)


def kernel(indices, table):
    idx3 = indices.reshape(NW, N_CHUNKS, CHUNK).astype(jnp.int32)
    out = _embed(idx3, table)
    return out.reshape(BATCH, SEQ, DIM)


# speedup vs baseline: 1.4454x; 1.3537x over previous
import jax
import jax.numpy as jnp
from jax import lax
from jax.experimental import pallas as pl
from jax.experimental.pallas import tpu as pltpu
from jax.experimental.pallas import tpu_sc as plsc

VOCAB = 100004
DIM = 200
BATCH = 4096
SEQ = 50

NC = 2
NS = 16
NW = NC * NS

B_TOTAL = BATCH * SEQ
ROWS_PER_W = B_TOTAL // NW
CHUNK = 100
N_CHUNKS = ROWS_PER_W // CHUNK
NBUF = 4
N_OUTER = N_CHUNKS // NBUF


def _body(table_hbm, idx_hbm, out_hbm, idx_v, *scratch):
    bufs = scratch[:NBUF]
    gsems = scratch[NBUF:2 * NBUF]
    wsems = scratch[2 * NBUF:]

    c = lax.axis_index("c")
    s = lax.axis_index("s")
    wid = s * NC + c
    base = wid * ROWS_PER_W

    pltpu.sync_copy(idx_hbm.at[wid], idx_v)

    for b in range(NBUF):
        pltpu.make_async_copy(
            table_hbm.at[idx_v.at[b]], bufs[b], gsems[b]).start()

    @pl.loop(0, N_OUTER)
    def _outer(o):
        c0 = o * NBUF
        for b in range(NBUF):
            ci = c0 + b
            pltpu.make_async_copy(
                table_hbm.at[idx_v.at[ci]], bufs[b], gsems[b]).wait()
            wb = pltpu.make_async_copy(
                bufs[b], out_hbm.at[pl.ds(base + ci * CHUNK, CHUNK)],
                wsems[b])
            wb.start()
            wb.wait()

            @pl.when(ci + NBUF < N_CHUNKS)
            def _():
                pltpu.make_async_copy(
                    table_hbm.at[idx_v.at[ci + NBUF]], bufs[b],
                    gsems[b]).start()


TBLK = 512


def _transpose_body(in_ref, out_ref):
    out_ref[...] = in_ref[...].T


@jax.jit
def _tc_transpose(t_t):
    return pl.pallas_call(
        _transpose_body,
        grid=(pl.cdiv(VOCAB, TBLK),),
        in_specs=[pl.BlockSpec((DIM, TBLK), lambda i: (0, i))],
        out_specs=pl.BlockSpec((TBLK, DIM), lambda i: (i, 0)),
        out_shape=jax.ShapeDtypeStruct((VOCAB, DIM), jnp.float32),
    )(t_t)


@jax.jit
def _embed(idx3, table):
    mesh = plsc.VectorSubcoreMesh(core_axis_name="c", subcore_axis_name="s")
    f = pl.kernel(
        _body,
        out_type=jax.ShapeDtypeStruct((B_TOTAL, DIM), jnp.float32),
        mesh=mesh,
        scratch_types=[
            pltpu.VMEM((N_CHUNKS, CHUNK), jnp.int32),
            *[pltpu.VMEM((CHUNK, DIM), jnp.float32) for _ in range(NBUF)],
            *[pltpu.SemaphoreType.DMA for _ in range(NBUF)],
            *[pltpu.SemaphoreType.DMA for _ in range(NBUF)],
        ],
        compiler_params=pltpu.CompilerParams(use_tc_tiling_on_sc=False),
    )
    return f(table, idx3)


def kernel(indices, table):
    table_r = _tc_transpose(table.T)
    idx3 = indices.reshape(NW, N_CHUNKS, CHUNK).astype(jnp.int32)
    out = _embed(idx3, table_r)
    return out.reshape(BATCH, SEQ, DIM)


# speedup vs baseline: 1.7207x; 1.1904x over previous
import jax
import jax.numpy as jnp
from jax import lax
from jax.experimental import pallas as pl
from jax.experimental.pallas import tpu as pltpu
from jax.experimental.pallas import tpu_sc as plsc

VOCAB = 100004
DIM = 200
BATCH = 4096
SEQ = 50

NC = 2
NS = 16
NW = NC * NS

B_TOTAL = BATCH * SEQ
ROWS_PER_W = B_TOTAL // NW
CHUNK = 100
N_CHUNKS = ROWS_PER_W // CHUNK
NBUF = 2
N_OUTER = N_CHUNKS // NBUF

TBLK = 512
DIM_A = 128
DIM_B = DIM - DIM_A


def _transpose_body(in_ref, out_a_ref, out_b_ref):
    blk = in_ref[...]
    out_a_ref[...] = blk[:DIM_A, :].T
    out_b_ref[:, :DIM_B] = blk[DIM_A:, :].T


@jax.jit
def _tc_transpose(t_t):
    return pl.pallas_call(
        _transpose_body,
        grid=(pl.cdiv(VOCAB, TBLK),),
        in_specs=[pl.BlockSpec((DIM, TBLK), lambda i: (0, i))],
        out_specs=[
            pl.BlockSpec((TBLK, DIM_A), lambda i: (i, 0)),
            pl.BlockSpec((TBLK, DIM_A), lambda i: (i, 0)),
        ],
        out_shape=[
            jax.ShapeDtypeStruct((VOCAB, DIM_A), jnp.float32),
            jax.ShapeDtypeStruct((VOCAB, DIM_A), jnp.float32),
        ],
    )(t_t)


def _body(ta_hbm, tb_hbm, idx_hbm, out_hbm, idx_v, *scratch):
    bufs_a = scratch[:NBUF]
    bufs_b = scratch[NBUF:2 * NBUF]
    gsems = scratch[2 * NBUF:3 * NBUF]
    wsems = scratch[3 * NBUF:]

    c = lax.axis_index("c")
    s = lax.axis_index("s")
    wid = s * NC + c
    base = wid * ROWS_PER_W

    pltpu.sync_copy(idx_hbm.at[wid], idx_v)

    def start_gathers(b, ci):
        pltpu.make_async_copy(
            ta_hbm.at[idx_v.at[ci]], bufs_a[b], gsems[b]).start()
        pltpu.make_async_copy(
            tb_hbm.at[idx_v.at[ci]], bufs_b[b], gsems[b]).start()

    def wait_gathers(b, ci):
        pltpu.make_async_copy(
            ta_hbm.at[idx_v.at[ci]], bufs_a[b], gsems[b]).wait()
        pltpu.make_async_copy(
            tb_hbm.at[idx_v.at[ci]], bufs_b[b], gsems[b]).wait()

    for b in range(NBUF):
        start_gathers(b, b)

    @pl.loop(0, N_OUTER)
    def _outer(o):
        c0 = o * NBUF
        for b in range(NBUF):
            ci = c0 + b
            rows = pl.ds(base + ci * CHUNK, CHUNK)
            wait_gathers(b, ci)
            wa = pltpu.make_async_copy(
                bufs_a[b], out_hbm.at[rows, pl.ds(0, DIM_A)], wsems[b])
            wb = pltpu.make_async_copy(
                bufs_b[b].at[:, pl.ds(0, DIM_B)],
                out_hbm.at[rows, pl.ds(DIM_A, DIM_B)], wsems[b])
            wa.start()
            wb.start()
            wa.wait()
            wb.wait()

            @pl.when(ci + NBUF < N_CHUNKS)
            def _():
                start_gathers(b, ci + NBUF)


@jax.jit
def _embed(idx3, ta, tb):
    mesh = plsc.VectorSubcoreMesh(core_axis_name="c", subcore_axis_name="s")
    f = pl.kernel(
        _body,
        out_type=jax.ShapeDtypeStruct((B_TOTAL, DIM), jnp.float32),
        mesh=mesh,
        scratch_types=[
            pltpu.VMEM((N_CHUNKS, CHUNK), jnp.int32),
            *[pltpu.VMEM((CHUNK, DIM_A), jnp.float32) for _ in range(2 * NBUF)],
            *[pltpu.SemaphoreType.DMA for _ in range(2 * NBUF)],
        ],
        compiler_params=pltpu.CompilerParams(use_tc_tiling_on_sc=False),
    )
    return f(ta, tb, idx3)


def kernel(indices, table):
    ta, tb = _tc_transpose(table.T)
    idx3 = indices.reshape(NW, N_CHUNKS, CHUNK).astype(jnp.int32)
    out = _embed(idx3, ta, tb)
    return out.reshape(BATCH, SEQ, DIM)
